# baseline (speedup 1.0000x reference)
import functools

import numpy as np
import jax
import jax.numpy as jnp
from jax import lax
from jax.experimental import pallas as pl
from jax.experimental.pallas import tpu as pltpu
from jax.experimental.pallas import tpu_sc as plsc

_D = 64
_BATCH = 4096
_SEQ = 200
_COND = 128
_MAXVAL = 100.0

_NC = 2
_NS = 16
_NW = _NC * _NS
_BPW = _BATCH // _NW
_NB = 8
_CH = _NB * _SEQ
_NCHUNK = _BPW // _NB

_BB = 512


def _tc_body(t_ref, cond_ref, w_ref, te_ref, co_ref):
    t = t_ref[:]
    col = lax.broadcasted_iota(jnp.int32, (_BB, _D), 1)
    half = jnp.where(col < _D // 2, col, col - _D // 2).astype(jnp.float32)
    inv_freq = jnp.exp(half * (-2.0 * float(np.log(_MAXVAL)) / _D))
    arg = t * inv_freq
    te_ref[:] = jnp.where(col < _D // 2, jnp.sin(arg), jnp.cos(arg))
    co_ref[:] = lax.dot_general(
        cond_ref[:], w_ref[:],
        dimension_numbers=(((1,), (1,)), ((), ())),
        preferred_element_type=jnp.float32,
    )


_tc_call = pl.pallas_call(
    _tc_body,
    grid=(_BATCH // _BB,),
    in_specs=[
        pl.BlockSpec((_BB, 1), lambda i: (i, 0)),
        pl.BlockSpec((_BB, _COND), lambda i: (i, 0)),
        pl.BlockSpec((_D, _COND), lambda i: (0, 0)),
    ],
    out_specs=[
        pl.BlockSpec((_BB, _D), lambda i: (i, 0)),
        pl.BlockSpec((_BB, _D), lambda i: (i, 0)),
    ],
    out_shape=[
        jax.ShapeDtypeStruct((_BATCH, _D), jnp.float32),
        jax.ShapeDtypeStruct((_BATCH, _D), jnp.float32),
    ],
)


def _sc_body(x_ref, te_ref, tab_ref, out_ref, idx_v, rows_v, tr_v, sem):
    wid = lax.axis_index("s") * _NC + lax.axis_index("c")
    b_base = wid * _BPW

    def chunk(ci, carry):
        b0 = b_base + ci * _NB
        pltpu.sync_copy(x_ref.at[pl.ds(b0 * _SEQ, _CH)], idx_v)
        pltpu.async_copy(tab_ref.at[idx_v], rows_v, sem).wait()
        pltpu.sync_copy(te_ref.at[pl.ds(b0, _NB)], tr_v)
        for i in range(_NB):
            r0 = (b0 + i) * (_SEQ + 1)
            pltpu.sync_copy(tr_v.at[pl.ds(i, 1)], out_ref.at[pl.ds(r0, 1)])
            pltpu.sync_copy(rows_v.at[pl.ds(i * _SEQ, _SEQ)],
                            out_ref.at[pl.ds(r0 + 1, _SEQ)])
        return carry

    lax.fori_loop(0, _NCHUNK, chunk, 0)


_sc_embed = pl.kernel(
    _sc_body,
    out_type=jax.ShapeDtypeStruct((_BATCH * (_SEQ + 1), _D), jnp.float32),
    mesh=plsc.VectorSubcoreMesh(core_axis_name="c", subcore_axis_name="s"),
    scratch_types=[
        pltpu.VMEM((_CH,), jnp.int32),
        pltpu.VMEM((_CH, _D), jnp.float32),
        pltpu.VMEM((_NB, _D), jnp.float32),
        pltpu.SemaphoreType.DMA,
    ],
    compiler_params=pltpu.CompilerParams(use_tc_tiling_on_sc=False),
)


@jax.jit
def kernel(x, t, condition_emb, x_emb_table, cond_weight):
    te, cond_out = _tc_call(t.reshape(_BATCH, 1), condition_emb, cond_weight)
    out = _sc_embed(x.reshape(_BATCH * _SEQ), te, x_emb_table)
    return out.reshape(_BATCH, _SEQ + 1, _D), cond_out


# speedup vs baseline: 1.0127x; 1.0127x over previous
import functools

import numpy as np
import jax
import jax.numpy as jnp
from jax import lax
from jax.experimental import pallas as pl
from jax.experimental.pallas import tpu as pltpu
from jax.experimental.pallas import tpu_sc as plsc

_D = 64
_BATCH = 4096
_SEQ = 200
_COND = 128
_MAXVAL = 100.0

_NC = 2
_NS = 16
_NW = _NC * _NS
_BPW = _BATCH // _NW
_NB = 8
_CH = _NB * _SEQ
_NCHUNK = _BPW // _NB

_BB = 512


def _tc_body(t_ref, cond_ref, w_ref, te_ref, co_ref):
    t = t_ref[:]
    col = lax.broadcasted_iota(jnp.int32, (_BB, _D), 1)
    half = jnp.where(col < _D // 2, col, col - _D // 2).astype(jnp.float32)
    inv_freq = jnp.exp(half * (-2.0 * float(np.log(_MAXVAL)) / _D))
    arg = t * inv_freq
    te_ref[:] = jnp.where(col < _D // 2, jnp.sin(arg), jnp.cos(arg))
    co_ref[:] = lax.dot_general(
        cond_ref[:], w_ref[:],
        dimension_numbers=(((1,), (1,)), ((), ())),
        preferred_element_type=jnp.float32,
    )


_tc_call = pl.pallas_call(
    _tc_body,
    grid=(_BATCH // _BB,),
    in_specs=[
        pl.BlockSpec((_BB, 1), lambda i: (i, 0)),
        pl.BlockSpec((_BB, _COND), lambda i: (i, 0)),
        pl.BlockSpec((_D, _COND), lambda i: (0, 0)),
    ],
    out_specs=[
        pl.BlockSpec((_BB, _D), lambda i: (i, 0)),
        pl.BlockSpec((_BB, _D), lambda i: (i, 0)),
    ],
    out_shape=[
        jax.ShapeDtypeStruct((_BATCH, _D), jnp.float32),
        jax.ShapeDtypeStruct((_BATCH, _D), jnp.float32),
    ],
)


_NB2 = 4
_CH2 = _NB2 * _SEQ
_ROWS = _NB2 * (_SEQ + 1)
_NCH2 = _BPW // _NB2
_NPAIR = _NCH2 // 2


def _sc_body(x_ref, te_ref, tab_ref, out_ref,
             idx0, idx1, buf0, buf1, gsem0, gsem1, wsem):
    wid = lax.axis_index("s") * _NC + lax.axis_index("c")
    b_base = wid * _BPW

    def gwait(buf, gsem):
        pltpu.make_async_copy(out_ref.at[pl.ds(0, _ROWS)], buf, gsem).wait()

    def wwait():
        pltpu.make_async_copy(buf0, out_ref.at[pl.ds(0, _ROWS)], wsem).wait()

    def fire_chunk(b0, idx, buf, gsem):
        pltpu.sync_copy(x_ref.at[pl.ds(b0 * _SEQ, _CH2)], idx)
        for i in range(_NB2):
            pltpu.async_copy(
                tab_ref.at[idx.at[pl.ds(i * _SEQ, _SEQ)]],
                buf.at[pl.ds(i * (_SEQ + 1) + 1, _SEQ)], gsem)
            pltpu.async_copy(
                te_ref.at[pl.ds(b0 + i, 1)],
                buf.at[pl.ds(i * (_SEQ + 1), 1)], gsem)

    def fire_write(b0, buf):
        pltpu.async_copy(buf, out_ref.at[pl.ds(b0 * (_SEQ + 1), _ROWS)], wsem)

    def pair(k, c):
        b0a = b_base + 2 * k * _NB2
        b0b = b0a + _NB2

        @pl.when(k >= 1)
        def _():
            wwait()
        fire_chunk(b0a, idx0, buf0, gsem0)

        @pl.when(k >= 1)
        def _():
            gwait(buf1, gsem1)
            fire_write(b0b - 2 * _NB2, buf1)
            wwait()

        fire_chunk(b0b, idx1, buf1, gsem1)
        gwait(buf0, gsem0)
        fire_write(b0a, buf0)
        return c

    lax.fori_loop(0, _NPAIR, pair, 0)
    gwait(buf1, gsem1)
    fire_write(b_base + (_NCH2 - 1) * _NB2, buf1)
    wwait()
    wwait()


_sc_embed = pl.kernel(
    _sc_body,
    out_type=jax.ShapeDtypeStruct((_BATCH * (_SEQ + 1), _D), jnp.float32),
    mesh=plsc.VectorSubcoreMesh(core_axis_name="c", subcore_axis_name="s"),
    scratch_types=[
        pltpu.VMEM((_CH2,), jnp.int32),
        pltpu.VMEM((_CH2,), jnp.int32),
        pltpu.VMEM((_ROWS, _D), jnp.float32),
        pltpu.VMEM((_ROWS, _D), jnp.float32),
        pltpu.SemaphoreType.DMA,
        pltpu.SemaphoreType.DMA,
        pltpu.SemaphoreType.DMA,
    ],
    compiler_params=pltpu.CompilerParams(use_tc_tiling_on_sc=False),
)


@jax.jit
def kernel(x, t, condition_emb, x_emb_table, cond_weight):
    te, cond_out = _tc_call(t.reshape(_BATCH, 1), condition_emb, cond_weight)
    out = _sc_embed(x.reshape(_BATCH * _SEQ), te, x_emb_table)
    return out.reshape(_BATCH, _SEQ + 1, _D), cond_out
